# Initial kernel scaffold; baseline (speedup 1.0000x reference)
import jax
import jax.numpy as jnp
from jax.experimental import pallas as pl


def kernel(x, deg, edge_index, walks, w_in, wq, wk, wv, wo, w1, w2, w_se, gcn1_w, gcn1_b, gcn2_w, gcn2_b):
    raise NotImplementedError("write your pallas kernel here")


# speedup vs baseline: 4.5019x; 4.5019x over previous
import functools

import jax
import jax.numpy as jnp
from jax import lax
from jax.experimental import pallas as pl
from jax.experimental.pallas import tpu as pltpu
from jax.experimental.pallas import tpu_sc as plsc

N = 10000
IN_DIM = 128
HID = 64
OUT = 64
NUM_WALKS = 4
WALK_LEN = 8
HEADS = 4
DH = HID // HEADS
N_EDGES = 320000
TOK = NUM_WALKS * WALK_LEN
NTOK = N * TOK
FW = 128

NPAD = 10240
NC, NS = 2, 16
NW = NC * NS
CHUNK = 80


def _t1_body(x_ref, w_ref, o_ref):
    o_ref[...] = jnp.dot(x_ref[...], w_ref[...],
                         preferred_element_type=jnp.float32)


def _project(x, w_in_pad):
    blk = 2000
    return pl.pallas_call(
        _t1_body,
        grid=(N // blk,),
        in_specs=[
            pl.BlockSpec((blk, IN_DIM), lambda i: (i, 0)),
            pl.BlockSpec((IN_DIM, FW), lambda i: (0, 0)),
        ],
        out_specs=pl.BlockSpec((blk, FW), lambda i: (i, 0)),
        out_shape=jax.ShapeDtypeStruct((N, FW), jnp.float32),
    )(x, w_in_pad)


def _gather_and_hist(xp, wflat, dst, ones_rows, zeros_rows):
    per_w = NTOK // NW
    nchunks = per_w // CHUNK
    rows_per_tile = NPAD // NS

    mesh = plsc.VectorSubcoreMesh(core_axis_name="c", subcore_axis_name="s")

    @functools.partial(
        pl.kernel, mesh=mesh,
        out_type=[
            jax.ShapeDtypeStruct((NTOK, FW), jnp.float32),
            jax.ShapeDtypeStruct((NC, NPAD, FW), jnp.float32),
        ],
        scratch_types=[
            pltpu.VMEM((CHUNK,), jnp.int32),
            pltpu.VMEM((CHUNK,), jnp.int32),
            pltpu.VMEM((CHUNK, FW), jnp.float32),
            pltpu.VMEM((CHUNK, FW), jnp.float32),
            pltpu.VMEM_SHARED((NPAD, FW), jnp.float32),
            pltpu.SemaphoreType.DMA,
        ],
    )
    def k(xp_hbm, idx_hbm, dst_hbm, ones_hbm, zeros_hbm, z_hbm, hist_hbm,
          idx_v, didx_v, rows_v, ones_v, acc_sh, sem):
        cid = lax.axis_index("c")
        sid = lax.axis_index("s")
        wid = cid * NS + sid
        rbase = pl.multiple_of(sid * rows_per_tile, 8)
        pltpu.sync_copy(zeros_hbm.at[pl.ds(0, rows_per_tile)],
                        acc_sh.at[pl.ds(rbase, rows_per_tile)])
        pltpu.sync_copy(ones_hbm, ones_v)
        plsc.subcore_barrier()

        base = pl.multiple_of(wid * per_w, 8)

        def body(j, _):
            off = pl.multiple_of(base + j * CHUNK, 8)
            pltpu.sync_copy(idx_hbm.at[pl.ds(off, CHUNK)], idx_v)
            pltpu.async_copy(xp_hbm.at[idx_v], rows_v, sem).wait()
            pltpu.sync_copy(rows_v, z_hbm.at[pl.ds(off, CHUNK)])
            pltpu.sync_copy(dst_hbm.at[pl.ds(off, CHUNK)], didx_v)
            pltpu.sync_copy(ones_v, acc_sh.at[didx_v], add=True)
            return 0

        lax.fori_loop(0, nchunks, body, 0)
        plsc.subcore_barrier()
        pltpu.sync_copy(acc_sh.at[pl.ds(rbase, rows_per_tile)],
                        hist_hbm.at[cid, pl.ds(rbase, rows_per_tile)])

    return k(xp, wflat, dst, ones_rows, zeros_rows)


def _edge_scatter(rows, src, dst, zeros_rows):
    per_w = N_EDGES // NW
    nchunks = per_w // CHUNK
    rows_per_tile = NPAD // NS

    mesh = plsc.VectorSubcoreMesh(core_axis_name="c", subcore_axis_name="s")

    @functools.partial(
        pl.kernel, mesh=mesh,
        out_type=jax.ShapeDtypeStruct((NC, NPAD, FW), jnp.float32),
        scratch_types=[
            pltpu.VMEM((CHUNK,), jnp.int32),
            pltpu.VMEM((CHUNK,), jnp.int32),
            pltpu.VMEM((CHUNK, FW), jnp.float32),
            pltpu.VMEM_SHARED((NPAD, FW), jnp.float32),
            pltpu.SemaphoreType.DMA,
        ],
    )
    def k(rows_hbm, src_hbm, dst_hbm, zeros_hbm, out_hbm,
          sidx_v, didx_v, rows_v, acc_sh, sem):
        cid = lax.axis_index("c")
        sid = lax.axis_index("s")
        wid = cid * NS + sid
        rbase = pl.multiple_of(sid * rows_per_tile, 8)
        pltpu.sync_copy(zeros_hbm.at[pl.ds(0, rows_per_tile)],
                        acc_sh.at[pl.ds(rbase, rows_per_tile)])
        plsc.subcore_barrier()

        base = pl.multiple_of(wid * per_w, 8)

        def body(j, _):
            off = pl.multiple_of(base + j * CHUNK, 8)
            pltpu.sync_copy(src_hbm.at[pl.ds(off, CHUNK)], sidx_v)
            pltpu.sync_copy(dst_hbm.at[pl.ds(off, CHUNK)], didx_v)
            pltpu.async_copy(rows_hbm.at[sidx_v], rows_v, sem).wait()
            pltpu.sync_copy(rows_v, acc_sh.at[didx_v], add=True)
            return 0

        lax.fori_loop(0, nchunks, body, 0)
        plsc.subcore_barrier()
        pltpu.sync_copy(acc_sh.at[pl.ds(rbase, rows_per_tile)],
                        out_hbm.at[cid, pl.ds(rbase, rows_per_tile)])

    return k(rows, src, dst, zeros_rows)


BN = 16
BT = BN * TOK


def _group_roll(arr, o, pos):
    t = arr.shape[0]
    a = jnp.concatenate([arr[o:], arr[:o]], axis=0)
    r2 = t + o - WALK_LEN
    b = jnp.concatenate([arr[r2:], arr[:r2]], axis=0)
    return jnp.where(pos < WALK_LEN - o, a, b)


def _t2_body(z_ref, deg_ref, hist_ref, wq_ref, wk_ref, wv_ref, wo_ref,
             w1_ref, w2_ref, wse_ref, g1_ref, hn1_ref, dinv_ref):
    f32 = jnp.float32
    z = z_ref[...][:, :HID]
    q = jnp.dot(z, wq_ref[...], preferred_element_type=f32) * (1.0 / 4.0)
    kk = jnp.dot(z, wk_ref[...], preferred_element_type=f32)
    v = jnp.dot(z, wv_ref[...], preferred_element_type=f32)

    di = lax.broadcasted_iota(jnp.int32, (HID, HEADS), 0) // DH
    hi = lax.broadcasted_iota(jnp.int32, (HID, HEADS), 1)
    E = (di == hi).astype(f32)
    pos = lax.broadcasted_iota(jnp.int32, (BT, HID), 0) % WALK_LEN

    ks = [kk] + [_group_roll(kk, o, pos) for o in range(1, WALK_LEN)]
    logits = [jnp.dot(q * ko, E, preferred_element_type=f32) for ko in ks]
    m = functools.reduce(jnp.maximum, logits)
    ws = [jnp.exp(s - m) for s in logits]
    den = functools.reduce(jnp.add, ws)
    inv_den = 1.0 / den
    o_acc = None
    for o in range(WALK_LEN):
        a_full = jnp.dot(ws[o] * inv_den, E.T, preferred_element_type=f32)
        vo = v if o == 0 else _group_roll(v, o, pos)
        contrib = a_full * vo
        o_acc = contrib if o_acc is None else o_acc + contrib
    z = z + jnp.dot(o_acc, wo_ref[...], preferred_element_type=f32)
    h1 = jnp.maximum(jnp.dot(z, w1_ref[...], preferred_element_type=f32), 0.0)
    z = z + jnp.dot(h1, w2_ref[...], preferred_element_type=f32)

    ri = lax.broadcasted_iota(jnp.int32, (BN, BT), 0)
    ci = lax.broadcasted_iota(jnp.int32, (BN, BT), 1) // TOK
    P = jnp.where(ri == ci, 1.0 / TOK, 0.0).astype(f32)
    pooled = jnp.dot(P, z, preferred_element_type=f32)

    deg = deg_ref[...]
    gf = 1.0 + jnp.log1p(jnp.maximum(deg, 0.0)) * wse_ref[...]
    gt = jnp.maximum(pooled * gf, 0.0)

    h = jnp.dot(gt, g1_ref[...], preferred_element_type=f32)
    degc = hist_ref[..., 0:1] + hist_ref[..., 1:2] + 1.0
    dinv = lax.rsqrt(degc)
    hn1_ref[...] = h * dinv
    dinv_ref[...] = dinv


def _transformer(z, deg2, hist2, wq, wk, wv, wo, w1, w2, wse2, gcn1_w):
    grid = (N // BN,)
    return pl.pallas_call(
        _t2_body,
        grid=grid,
        in_specs=[
            pl.BlockSpec((BT, FW), lambda i: (i, 0)),
            pl.BlockSpec((BN, 1), lambda i: (i, 0)),
            pl.BlockSpec((BN, 2), lambda i: (i, 0)),
            pl.BlockSpec((HID, HID), lambda i: (0, 0)),
            pl.BlockSpec((HID, HID), lambda i: (0, 0)),
            pl.BlockSpec((HID, HID), lambda i: (0, 0)),
            pl.BlockSpec((HID, HID), lambda i: (0, 0)),
            pl.BlockSpec((HID, 2 * HID), lambda i: (0, 0)),
            pl.BlockSpec((2 * HID, HID), lambda i: (0, 0)),
            pl.BlockSpec((1, HID), lambda i: (0, 0)),
            pl.BlockSpec((HID, 2 * OUT), lambda i: (0, 0)),
        ],
        out_specs=[
            pl.BlockSpec((BN, 2 * OUT), lambda i: (i, 0)),
            pl.BlockSpec((BN, 1), lambda i: (i, 0)),
        ],
        out_shape=[
            jax.ShapeDtypeStruct((N, 2 * OUT), jnp.float32),
            jax.ShapeDtypeStruct((N, 1), jnp.float32),
        ],
    )(z, deg2, hist2, wq, wk, wv, wo, w1, w2, wse2, gcn1_w)


def _t3_body(p0_ref, p1_ref, hn1_ref, dinv_ref, b1_ref, g2_ref, hn2_ref):
    dinv = dinv_ref[...]
    s = p0_ref[...] + p1_ref[...] + hn1_ref[...]
    h1 = jnp.maximum(dinv * s + b1_ref[...], 0.0)
    hn2_ref[...] = dinv * jnp.dot(h1, g2_ref[...],
                                  preferred_element_type=jnp.float32)


def _gcn_mid(p0, p1, hn1, dinv, b1_2, gcn2_w_pad):
    blk = 2000
    return pl.pallas_call(
        _t3_body,
        grid=(N // blk,),
        in_specs=[
            pl.BlockSpec((blk, FW), lambda i: (i, 0)),
            pl.BlockSpec((blk, FW), lambda i: (i, 0)),
            pl.BlockSpec((blk, 2 * OUT), lambda i: (i, 0)),
            pl.BlockSpec((blk, 1), lambda i: (i, 0)),
            pl.BlockSpec((1, 2 * OUT), lambda i: (0, 0)),
            pl.BlockSpec((2 * OUT, FW), lambda i: (0, 0)),
        ],
        out_specs=pl.BlockSpec((blk, FW), lambda i: (i, 0)),
        out_shape=jax.ShapeDtypeStruct((N, FW), jnp.float32),
    )(p0, p1, hn1, dinv, b1_2, gcn2_w_pad)


def _t4_body(p0_ref, p1_ref, hn2_ref, dinv_ref, b2_ref, o_ref):
    s = p0_ref[...] + p1_ref[...] + hn2_ref[...]
    s = dinv_ref[...] * s[:, :OUT] + b2_ref[...]
    m = jnp.max(s, axis=1, keepdims=True)
    e = jnp.exp(s - m)
    o_ref[...] = e / jnp.sum(e, axis=1, keepdims=True)


def _finalize(p0, p1, hn2, dinv, b2_2):
    blk = 2000
    return pl.pallas_call(
        _t4_body,
        grid=(N // blk,),
        in_specs=[
            pl.BlockSpec((blk, FW), lambda i: (i, 0)),
            pl.BlockSpec((blk, FW), lambda i: (i, 0)),
            pl.BlockSpec((blk, FW), lambda i: (i, 0)),
            pl.BlockSpec((blk, 1), lambda i: (i, 0)),
            pl.BlockSpec((1, OUT), lambda i: (0, 0)),
        ],
        out_specs=pl.BlockSpec((blk, OUT), lambda i: (i, 0)),
        out_shape=jax.ShapeDtypeStruct((N, OUT), jnp.float32),
    )(p0, p1, hn2, dinv, b2_2)


def kernel(x, deg, edge_index, walks, w_in, wq, wk, wv, wo, w1, w2, w_se,
           gcn1_w, gcn1_b, gcn2_w, gcn2_b):
    f32 = jnp.float32
    wflat = walks.reshape(-1).astype(jnp.int32)
    src = edge_index[0].astype(jnp.int32)
    dst = edge_index[1].astype(jnp.int32)

    ones_rows = jnp.ones((CHUNK, FW), f32)
    zeros_rows = jnp.zeros((NPAD // NS, FW), f32)

    w_in_pad = jnp.pad(w_in, ((0, 0), (0, FW - HID)))
    xp = _project(x, w_in_pad)
    z, hist = _gather_and_hist(xp, wflat, dst, ones_rows, zeros_rows)
    hist2 = hist[:, :N, 0].T

    hn1, dinv = _transformer(
        z, deg.reshape(N, 1), hist2, wq[0], wk[0], wv[0], wo[0],
        w1[0], w2[0], w_se.reshape(1, HID), gcn1_w)

    s1 = _edge_scatter(hn1, src, dst, zeros_rows)
    gcn2_w_pad = jnp.pad(gcn2_w, ((0, 0), (0, FW - OUT)))
    hn2 = _gcn_mid(s1[0, :N], s1[1, :N], hn1, dinv,
                   gcn1_b.reshape(1, 2 * OUT), gcn2_w_pad)

    s2 = _edge_scatter(hn2, src, dst, zeros_rows)
    return _finalize(s2[0, :N], s2[1, :N], hn2, dinv,
                     gcn2_b.reshape(1, OUT))
